# Initial kernel scaffold; baseline (speedup 1.0000x reference)
import jax
import jax.numpy as jnp
from jax.experimental import pallas as pl


def kernel(x, edge_index, W1_l, W1_r, b1, W2_l, W2_r, b2):
    raise NotImplementedError("write your pallas kernel here")


# speedup vs baseline: 7.9542x; 7.9542x over previous
import functools

import jax
import jax.numpy as jnp
from jax import lax
from jax.experimental import pallas as pl
from jax.experimental.pallas import tpu as pltpu
from jax.experimental.pallas import tpu_sc as plsc

NC = 2
NS = 16
NW = NC * NS

DW = 128
EB = 128
KB = 8


def _make_edge_pass(n_acc, n_chunk_rows):
    rows_per_w = n_chunk_rows // NW
    n_outer = rows_per_w // KB
    stripe = n_acc // NS

    mesh = plsc.VectorSubcoreMesh(core_axis_name="c", subcore_axis_name="s")

    @functools.partial(
        pl.kernel,
        out_type=[jax.ShapeDtypeStruct((NC, n_acc, DW), jnp.float32)],
        mesh=mesh,
        scratch_types=[
            pltpu.VMEM((KB, EB), jnp.int32),
            pltpu.VMEM((KB, EB), jnp.int32),
            pltpu.VMEM((EB, DW), jnp.float32),
            pltpu.SemaphoreType.DMA,
            pltpu.VMEM_SHARED((n_acc, DW), jnp.float32),
        ],
    )
    def edge_pass(table, src, dst, zacc, acc_out,
                  src_v, dst_v, rows_v, gsem, acc_sh):
        c = lax.axis_index("c")
        s = lax.axis_index("s")
        wid = s * NC + c
        st = s * stripe
        pltpu.sync_copy(zacc.at[pl.ds(st, stripe)],
                        acc_sh.at[pl.ds(st, stripe)])
        plsc.subcore_barrier()

        row0 = wid * rows_per_w

        def outer(it, carry):
            r = row0 + it * KB
            pltpu.sync_copy(src.at[pl.ds(r, KB)], src_v)
            pltpu.sync_copy(dst.at[pl.ds(r, KB)], dst_v)
            for j in range(KB):
                pltpu.async_copy(table.at[src_v.at[j]], rows_v, gsem).wait()
                pltpu.sync_copy(rows_v, acc_sh.at[dst_v.at[j]], add=True)
            return carry

        lax.fori_loop(0, n_outer, outer, 0)

        plsc.subcore_barrier()
        pltpu.sync_copy(acc_sh.at[pl.ds(st, stripe)],
                        acc_out.at[c, pl.ds(st, stripe)])

    return edge_pass


def _layer1_pre(x, w1, d_hid, bm):
    n, k = x.shape
    dw2 = 2 * d_hid

    def body(x_ref, w_ref, t_ref, r_ref):
        z = jnp.dot(x_ref[...], w_ref[...], preferred_element_type=jnp.float32)
        ones = jnp.ones((bm, 1), jnp.float32)
        zeros = jnp.zeros((bm, DW - d_hid - 1), jnp.float32)
        t_ref[...] = jnp.concatenate([z[:, :d_hid], ones, zeros], axis=1)
        r_ref[...] = z[:, d_hid:]

    return pl.pallas_call(
        body,
        grid=(n // bm,),
        in_specs=[pl.BlockSpec((bm, k), lambda i: (i, 0)),
                  pl.BlockSpec((k, dw2), lambda i: (0, 0))],
        out_specs=[pl.BlockSpec((bm, DW), lambda i: (i, 0)),
                   pl.BlockSpec((bm, d_hid), lambda i: (i, 0))],
        out_shape=[jax.ShapeDtypeStruct((n, DW), jnp.float32),
                   jax.ShapeDtypeStruct((n, d_hid), jnp.float32)],
    )(x, w1)


def _layer1_post(a0, a1, r1, b1, bm):
    n, d = r1.shape

    def body(a0_ref, a1_ref, r1_ref, b1_ref, h_ref):
        a = a0_ref[...] + a1_ref[...]
        deg = jnp.maximum(a[:, d:d + 1], 1.0)
        mean = a[:, :d] / deg
        h = jnp.maximum(mean + r1_ref[...] + b1_ref[...], 0.0)
        h_ref[...] = jnp.concatenate(
            [h, jnp.zeros((bm, DW - d), jnp.float32)], axis=1)

    return pl.pallas_call(
        body,
        grid=(n // bm,),
        in_specs=[pl.BlockSpec((bm, DW), lambda i: (i, 0)),
                  pl.BlockSpec((bm, DW), lambda i: (i, 0)),
                  pl.BlockSpec((bm, d), lambda i: (i, 0)),
                  pl.BlockSpec((1, d), lambda i: (0, 0))],
        out_specs=pl.BlockSpec((bm, DW), lambda i: (i, 0)),
        out_shape=jax.ShapeDtypeStruct((n, DW), jnp.float32),
    )(a0, a1, r1, b1)


def _layer2_post(a0, a1, g0, g1, h, w_l, w_r, b2, bm):
    d = w_l.shape[0]
    m = w_l.shape[1]
    n = h.shape[0]

    def body(a0_ref, a1_ref, g0_ref, g1_ref, h_ref, wl_ref, wr_ref, b2_ref,
             o_ref):
        deg = jnp.maximum(g0_ref[...] + g1_ref[...], 1.0)
        mean = (a0_ref[...] + a1_ref[...])[:, :d] / deg
        o_ref[...] = (jnp.dot(mean, wl_ref[...],
                              preferred_element_type=jnp.float32)
                      + jnp.dot(h_ref[...][:, :d], wr_ref[...],
                                preferred_element_type=jnp.float32)
                      + b2_ref[...])

    return pl.pallas_call(
        body,
        grid=(n // bm,),
        in_specs=[pl.BlockSpec((bm, DW), lambda i: (i, 0)),
                  pl.BlockSpec((bm, DW), lambda i: (i, 0)),
                  pl.BlockSpec((bm, 1), lambda i: (i, 0)),
                  pl.BlockSpec((bm, 1), lambda i: (i, 0)),
                  pl.BlockSpec((bm, DW), lambda i: (i, 0)),
                  pl.BlockSpec((d, m), lambda i: (0, 0)),
                  pl.BlockSpec((d, m), lambda i: (0, 0)),
                  pl.BlockSpec((1, m), lambda i: (0, 0))],
        out_specs=pl.BlockSpec((bm, m), lambda i: (i, 0)),
        out_shape=jax.ShapeDtypeStruct((n, m), jnp.float32),
    )(a0, a1, g0, g1, h, w_l, w_r, b2)


def kernel(x, edge_index, W1_l, W1_r, b1, W2_l, W2_r, b2):
    n, d_in = x.shape
    d_hid = W1_l.shape[1]
    d_out = W2_l.shape[1]
    e = edge_index.shape[1]

    rpw = -(-e // (NW * EB))
    rpw = -(-rpw // KB) * KB
    n_chunk_rows = NW * rpw
    e_pad = n_chunk_rows * EB

    n_acc = -(-(n + 1) // (NS * 8)) * (NS * 8)

    src = edge_index[0].astype(jnp.int32)
    dst = edge_index[1].astype(jnp.int32)
    pad_i = jnp.arange(e_pad - e, dtype=jnp.int32)
    src_p = jnp.concatenate([src, pad_i % n]).reshape(n_chunk_rows, EB)
    dst_p = jnp.concatenate(
        [dst, n + pad_i % (n_acc - n)]).reshape(n_chunk_rows, EB)

    zacc = jnp.zeros((n_acc, DW), jnp.float32)

    w1 = jnp.concatenate([W1_l, W1_r], axis=1)
    table1, r1 = _layer1_pre(x, w1, d_hid, bm=1000)
    table1 = jnp.concatenate(
        [table1, jnp.zeros((n_acc - n, DW), jnp.float32)])

    edge_pass = _make_edge_pass(n_acc, n_chunk_rows)
    (acc1,) = edge_pass(table1, src_p, dst_p, zacc)

    h128 = _layer1_post(acc1[0][:n], acc1[1][:n], r1,
                        b1.reshape(1, d_hid), bm=1000)

    h128p = jnp.concatenate([h128, jnp.zeros((n_acc - n, DW), jnp.float32)])
    (acc2,) = edge_pass(h128p, src_p, dst_p, zacc)

    g0 = acc1[0][:n, d_hid:d_hid + 1]
    g1 = acc1[1][:n, d_hid:d_hid + 1]
    out = _layer2_post(acc2[0][:n], acc2[1][:n], g0, g1, h128,
                       W2_l, W2_r, b2.reshape(1, d_out), bm=1000)
    return out


# speedup vs baseline: 12.6528x; 1.5907x over previous
import functools

import jax
import jax.numpy as jnp
from jax import lax
from jax.experimental import pallas as pl
from jax.experimental.pallas import tpu as pltpu
from jax.experimental.pallas import tpu_sc as plsc

NC = 2
NS = 16
NW = NC * NS

DW = 128
EB = 128


def _make_edge_pass(n_acc, n_chunk_rows):
    rpw = n_chunk_rows // NW
    nh = 2
    rph = rpw // nh
    stripe = n_acc // NS

    mesh = plsc.VectorSubcoreMesh(core_axis_name="c", subcore_axis_name="s")

    @functools.partial(
        pl.kernel,
        out_type=[jax.ShapeDtypeStruct((NC, n_acc, DW), jnp.float32)],
        mesh=mesh,
        scratch_types=[
            pltpu.VMEM((rph, EB), jnp.int32),
            pltpu.VMEM((rph, EB), jnp.int32),
            pltpu.VMEM((EB, DW), jnp.float32),
            pltpu.VMEM((EB, DW), jnp.float32),
            pltpu.SemaphoreType.DMA,
            pltpu.SemaphoreType.DMA,
            pltpu.VMEM_SHARED((n_acc, DW), jnp.float32),
        ],
    )
    def edge_pass(table, src, dst, zacc, acc_out,
                  src_v, dst_v, rows0, rows1, g0, g1, acc_sh):
        c = lax.axis_index("c")
        s = lax.axis_index("s")
        wid = s * NC + c
        st = s * stripe
        pltpu.sync_copy(zacc.at[pl.ds(st, stripe)],
                        acc_sh.at[pl.ds(st, stripe)])
        plsc.subcore_barrier()

        row0 = wid * rpw

        def half(hh, carry):
            base = row0 + hh * rph
            pltpu.sync_copy(src.at[pl.ds(base, rph)], src_v)
            pltpu.sync_copy(dst.at[pl.ds(base, rph)], dst_v)
            pltpu.async_copy(table.at[src_v.at[0]], rows0, g0)

            def pair(it, c2):
                b0 = it * 2
                pltpu.async_copy(table.at[src_v.at[b0 + 1]], rows1, g1)
                pltpu.make_async_copy(table.at[src_v.at[b0]], rows0, g0).wait()
                pltpu.sync_copy(rows0, acc_sh.at[dst_v.at[b0]], add=True)

                @pl.when(b0 + 2 < rph)
                def _fire_next():
                    pltpu.async_copy(table.at[src_v.at[b0 + 2]], rows0, g0)

                pltpu.make_async_copy(table.at[src_v.at[b0 + 1]],
                                      rows1, g1).wait()
                pltpu.sync_copy(rows1, acc_sh.at[dst_v.at[b0 + 1]], add=True)
                return c2

            lax.fori_loop(0, rph // 2, pair, 0)
            return carry

        lax.fori_loop(0, nh, half, 0)

        plsc.subcore_barrier()
        pltpu.sync_copy(acc_sh.at[pl.ds(st, stripe)],
                        acc_out.at[c, pl.ds(st, stripe)])

    return edge_pass


def _layer1_pre(x, w1, d_hid, bm):
    n, k = x.shape
    dw2 = 2 * d_hid

    def body(x_ref, w_ref, t_ref, r_ref):
        z = jnp.dot(x_ref[...], w_ref[...], preferred_element_type=jnp.float32)
        ones = jnp.ones((bm, 1), jnp.float32)
        zeros = jnp.zeros((bm, DW - d_hid - 1), jnp.float32)
        t_ref[...] = jnp.concatenate([z[:, :d_hid], ones, zeros], axis=1)
        r_ref[...] = z[:, d_hid:]

    return pl.pallas_call(
        body,
        grid=(n // bm,),
        in_specs=[pl.BlockSpec((bm, k), lambda i: (i, 0)),
                  pl.BlockSpec((k, dw2), lambda i: (0, 0))],
        out_specs=[pl.BlockSpec((bm, DW), lambda i: (i, 0)),
                   pl.BlockSpec((bm, d_hid), lambda i: (i, 0))],
        out_shape=[jax.ShapeDtypeStruct((n, DW), jnp.float32),
                   jax.ShapeDtypeStruct((n, d_hid), jnp.float32)],
    )(x, w1)


def _layer1_post(acc1, r1, b1, bm):
    n, d = r1.shape

    def body(a_ref, r1_ref, b1_ref, h_ref):
        a = a_ref[0] + a_ref[1]
        deg = jnp.maximum(a[:, d:d + 1], 1.0)
        h = jnp.maximum(a[:, :d] / deg + r1_ref[...] + b1_ref[...], 0.0)
        h_ref[...] = jnp.concatenate(
            [h, jnp.zeros((bm, DW - d), jnp.float32)], axis=1)

    return pl.pallas_call(
        body,
        grid=(n // bm,),
        in_specs=[pl.BlockSpec((2, bm, DW), lambda i: (0, i, 0)),
                  pl.BlockSpec((bm, d), lambda i: (i, 0)),
                  pl.BlockSpec((1, d), lambda i: (0, 0))],
        out_specs=pl.BlockSpec((bm, DW), lambda i: (i, 0)),
        out_shape=jax.ShapeDtypeStruct((n, DW), jnp.float32),
    )(acc1, r1, b1)


def _layer2_post(acc2, acc1, h, w_l, w_r, b2, bm):
    d, m = w_l.shape
    n = h.shape[0]

    def body(a2_ref, a1_ref, h_ref, wl_ref, wr_ref, b2_ref, o_ref):
        a1 = a1_ref[0] + a1_ref[1]
        deg = jnp.maximum(a1[:, d:d + 1], 1.0)
        mean = (a2_ref[0] + a2_ref[1])[:, :d] / deg
        o_ref[...] = (jnp.dot(mean, wl_ref[...],
                              preferred_element_type=jnp.float32)
                      + jnp.dot(h_ref[...][:, :d], wr_ref[...],
                                preferred_element_type=jnp.float32)
                      + b2_ref[...])

    return pl.pallas_call(
        body,
        grid=(n // bm,),
        in_specs=[pl.BlockSpec((2, bm, DW), lambda i: (0, i, 0)),
                  pl.BlockSpec((2, bm, DW), lambda i: (0, i, 0)),
                  pl.BlockSpec((bm, DW), lambda i: (i, 0)),
                  pl.BlockSpec((d, m), lambda i: (0, 0)),
                  pl.BlockSpec((d, m), lambda i: (0, 0)),
                  pl.BlockSpec((1, m), lambda i: (0, 0))],
        out_specs=pl.BlockSpec((bm, m), lambda i: (i, 0)),
        out_shape=jax.ShapeDtypeStruct((n, m), jnp.float32),
    )(acc2, acc1, h, w_l, w_r, b2)


def kernel(x, edge_index, W1_l, W1_r, b1, W2_l, W2_r, b2):
    n, d_in = x.shape
    d_hid = W1_l.shape[1]
    d_out = W2_l.shape[1]
    e = edge_index.shape[1]

    rpw = -(-e // (NW * EB))
    rpw = -(-rpw // 4) * 4
    n_chunk_rows = NW * rpw
    e_pad = n_chunk_rows * EB

    n_acc = -(-(n + 1) // (NS * 8)) * (NS * 8)

    src = edge_index[0].astype(jnp.int32)
    dst = edge_index[1].astype(jnp.int32)
    pad_i = jnp.arange(e_pad - e, dtype=jnp.int32)
    src_p = jnp.concatenate([src, pad_i % n]).reshape(n_chunk_rows, EB)
    dst_p = jnp.concatenate(
        [dst, n + pad_i % (n_acc - n)]).reshape(n_chunk_rows, EB)

    zacc = jnp.zeros((n_acc, DW), jnp.float32)

    w1 = jnp.concatenate([W1_l, W1_r], axis=1)
    table1, r1 = _layer1_pre(x, w1, d_hid, bm=1000)

    edge_pass = _make_edge_pass(n_acc, n_chunk_rows)
    (acc1,) = edge_pass(table1, src_p, dst_p, zacc)

    h128 = _layer1_post(acc1, r1, b1.reshape(1, d_hid), bm=1000)

    (acc2,) = edge_pass(h128, src_p, dst_p, zacc)

    out = _layer2_post(acc2, acc1, h128,
                       W2_l, W2_r, b2.reshape(1, d_out), bm=1000)
    return out
